# baseline (speedup 1.0000x reference)
import jax
import jax.numpy as jnp
from jax import lax
from jax.experimental import pallas as pl
from jax.experimental.pallas import tpu as pltpu
from jax.experimental.pallas import tpu_sc as plsc

NUM_CORES = 2
NUM_SUBCORES = 16
LANES = 16
NUM_WORKERS = NUM_CORES * NUM_SUBCORES

BATCH = 16384
EMBED = 16
B_PER_W = BATCH // NUM_WORKERS
CHUNK = 128
NCHUNKS = B_PER_W // CHUNK
NGROUPS = B_PER_W // LANES

_BITREV = [0, 8, 4, 12, 2, 10, 6, 14, 1, 9, 5, 13, 3, 11, 7, 15]


def _vperm(v, idx):
    return jnp.take_along_axis(v, idx, axis=0)


def _body(uid_hbm, iid_hbm, uemb_hbm, iemb_hbm, ubias_hbm, ibias_hbm,
          out_hbm, uidx_v, iidx_v, urows_v, irows_v, ub_v, ib_v, out_v,
          sem):
    wid = lax.axis_index("s") * NUM_CORES + lax.axis_index("c")
    base = wid * B_PER_W

    for j in range(NCHUNKS):
        pltpu.sync_copy(uid_hbm.at[pl.ds(base + j * CHUNK, CHUNK)],
                        uidx_v.at[j])
        pltpu.sync_copy(iid_hbm.at[pl.ds(base + j * CHUNK, CHUNK)],
                        iidx_v.at[j])

    copies = []
    for j in range(NCHUNKS):
        lo = j * CHUNK
        copies.append(pltpu.async_copy(
            uemb_hbm.at[uidx_v.at[j]], urows_v.at[pl.ds(lo, CHUNK)], sem))
        copies.append(pltpu.async_copy(
            iemb_hbm.at[iidx_v.at[j]], irows_v.at[pl.ds(lo, CHUNK)], sem))
        copies.append(pltpu.async_copy(
            ubias_hbm.at[uidx_v.at[j]], ub_v.at[pl.ds(lo, CHUNK)], sem))
        copies.append(pltpu.async_copy(
            ibias_hbm.at[iidx_v.at[j]], ib_v.at[pl.ds(lo, CHUNK)], sem))
    for c in copies:
        c.wait()

    iota = lax.iota(jnp.int32, LANES)
    bitrev = (((iota & 1) << 3) | ((iota & 2) << 1)
              | ((iota & 4) >> 1) | ((iota & 8) >> 3))
    masks = {h: (iota & h) == 0 for h in (8, 4, 2, 1)}
    perms = {h: iota ^ h for h in (8, 4, 2, 1)}

    def combine(a, b, h):
        r1 = jnp.where(masks[h], a, b)
        r2 = jnp.where(masks[h], b, a)
        return r1 + _vperm(r2, perms[h])

    def group(g, carry):
        base_r = g * LANES
        vecs = []
        for j in range(LANES):
            u = urows_v[base_r + j]
            v = irows_v[base_r + j]
            vecs.append(u * v)
        for h in (8, 4, 2, 1):
            vecs = [combine(vecs[2 * t], vecs[2 * t + 1], h)
                    for t in range(len(vecs) // 2)]
        z = vecs[0]
        ridx = base_r + bitrev
        ub = plsc.load_gather(ub_v, [ridx])
        ib = plsc.load_gather(ib_v, [ridx])
        x = z + ub + ib
        x = jnp.maximum(x, 0.0)
        e2 = jnp.exp(-2.0 * x)
        y = (1.0 - e2) / (1.0 + e2)
        plsc.store_scatter(out_v, [ridx], y)
        return carry

    lax.fori_loop(0, NGROUPS, group, 0)

    pltpu.sync_copy(out_v, out_hbm.at[pl.ds(base, B_PER_W)])


_sc_kernel = pl.kernel(
    _body,
    out_type=jax.ShapeDtypeStruct((BATCH,), jnp.float32),
    mesh=plsc.VectorSubcoreMesh(core_axis_name="c", subcore_axis_name="s"),
    compiler_params=pltpu.CompilerParams(
        needs_layout_passes=False, use_tc_tiling_on_sc=False),
    scratch_types=[
        pltpu.VMEM((NCHUNKS, CHUNK), jnp.int32),
        pltpu.VMEM((NCHUNKS, CHUNK), jnp.int32),
        pltpu.VMEM((B_PER_W, EMBED), jnp.float32),
        pltpu.VMEM((B_PER_W, EMBED), jnp.float32),
        pltpu.VMEM((B_PER_W,), jnp.float32),
        pltpu.VMEM((B_PER_W,), jnp.float32),
        pltpu.VMEM((B_PER_W,), jnp.float32),
        pltpu.SemaphoreType.DMA,
    ],
)


@jax.jit
def kernel(inputs, user_emb, item_emb, user_bias, item_bias):
    uid = inputs[:, 0].astype(jnp.int32)
    iid = inputs[:, 1].astype(jnp.int32)
    out = _sc_kernel(uid, iid, user_emb, item_emb,
                     user_bias.reshape(-1), item_bias.reshape(-1))
    return out.reshape(BATCH, 1)
